# Initial kernel scaffold; baseline (speedup 1.0000x reference)
import jax
import jax.numpy as jnp
from jax.experimental import pallas as pl


def kernel(symbols_prob, dfa_table):
    raise NotImplementedError("write your pallas kernel here")


# speedup vs baseline: 34.7312x; 34.7312x over previous
import functools

import jax
import jax.numpy as jnp
from jax import lax
from jax.experimental import pallas as pl
from jax.experimental.pallas import tpu as pltpu
from jax.experimental.pallas import tpu_sc as plsc

S = 4096
SYM = 256
T = 256
NS = 16
SRC = S // NS
L = 16


def _fuzzy_body(sym_hbm, dfa_hbm, out_hbm,
                dfa_v, act_v, st_v, acc_v, red_v, sym_sh, part_sh):
    sid = lax.axis_index("s")
    base = sid * SRC

    pltpu.sync_copy(dfa_hbm.at[pl.ds(base * SYM, SRC * SYM)], dfa_v)

    @pl.when(sid == 0)
    def _():
        pltpu.sync_copy(sym_hbm, sym_sh)

    def zero_state(i, c):
        st_v[pl.ds(i * L, L)] = jnp.zeros((L,), jnp.float32)
        return c
    lax.fori_loop(0, SRC // L, zero_state, None)

    @pl.when(sid == 0)
    def _():
        lane = lax.iota(jnp.int32, L)
        st_v[pl.ds(0, L)] = jnp.where(lane == 0,
                                      jnp.float32(1.0), jnp.float32(0.0))

    plsc.subcore_barrier()

    def step(t, carry):
        pltpu.sync_copy(sym_sh.at[t], act_v)

        def zero_acc(i, c):
            acc_v[pl.ds(i * L, L)] = jnp.zeros((L,), jnp.float32)
            return c
        lax.fori_loop(0, S // L, zero_acc, None, unroll=8)

        avec = [act_v[pl.ds(k * L, L)] for k in range(SYM // L)]

        def grp_body(g, c):
            sv = st_v[pl.ds(g * L, L)]
            for j in range(L):
                stv = sv[j]
                off = (g * L + j) * SYM
                for k in range(SYM // L):
                    idx = dfa_v[pl.ds(off + k * L, L)]
                    plsc.addupdate_scatter(acc_v, [idx], stv * avec[k])
            return c
        lax.fori_loop(0, SRC // L, grp_body, None)

        pltpu.sync_copy(acc_v, part_sh.at[sid])
        plsc.subcore_barrier()
        pltpu.sync_copy(part_sh.at[:, pl.ds(base, SRC)], red_v)

        def red_body(i, c):
            tot = red_v[0, pl.ds(i * L, L)]
            for r in range(1, NS):
                tot = tot + red_v[r, pl.ds(i * L, L)]
            st_v[pl.ds(i * L, L)] = tot
            return c
        lax.fori_loop(0, SRC // L, red_body, None)
        plsc.subcore_barrier()
        return carry

    lax.fori_loop(0, T, step, None)

    pltpu.sync_copy(st_v, out_hbm.at[pl.ds(base, SRC)])


@jax.jit
def kernel(symbols_prob, dfa_table):
    dfa_flat = dfa_table.reshape(-1)
    mesh = plsc.VectorSubcoreMesh(core_axis_name="c", subcore_axis_name="s",
                                  num_cores=1, num_subcores=NS)
    run = pl.kernel(
        _fuzzy_body,
        out_type=jax.ShapeDtypeStruct((S,), jnp.float32),
        mesh=mesh,
        compiler_params=pltpu.CompilerParams(needs_layout_passes=False),
        scratch_types=[
            pltpu.VMEM((SRC * SYM,), jnp.int32),
            pltpu.VMEM((SYM,), jnp.float32),
            pltpu.VMEM((SRC,), jnp.float32),
            pltpu.VMEM((S,), jnp.float32),
            pltpu.VMEM((NS, SRC), jnp.float32),
            pltpu.VMEM_SHARED((T, SYM), jnp.float32),
            pltpu.VMEM_SHARED((NS, S), jnp.float32),
        ],
    )
    return run(symbols_prob, dfa_flat)


# speedup vs baseline: 112.1430x; 3.2289x over previous
import functools

import jax
import jax.numpy as jnp
from jax import lax
from jax.experimental import pallas as pl
from jax.experimental.pallas import tpu as pltpu
from jax.experimental.pallas import tpu_sc as plsc

S = 4096
SYM = 256
T = 256
NS = 16
SRC = S // NS
L = 16
PW = SRC * SYM // 2


def _fuzzy_body(sym_hbm, dfa_hbm, out_hbm,
                dfa_v, sym_v, st_v, acc_v, red_v, part_sh):
    sid = lax.axis_index("s")
    base = sid * SRC

    pltpu.sync_copy(dfa_hbm.at[pl.ds(sid * PW, PW)], dfa_v)
    pltpu.sync_copy(sym_hbm, sym_v)

    def zero_state(i, c):
        st_v[pl.ds(i * L, L)] = jnp.zeros((L,), jnp.float32)
        return c
    lax.fori_loop(0, SRC // L, zero_state, None)

    @pl.when(sid == 0)
    def _():
        lane = lax.iota(jnp.int32, L)
        st_v[pl.ds(0, L)] = jnp.where(lane == 0,
                                      jnp.float32(1.0), jnp.float32(0.0))

    @plsc.parallel_loop(0, S // L, unroll=8)
    def zero_acc0(i):
        acc_v[pl.ds(i * L, L)] = jnp.zeros((L,), jnp.float32)

    plsc.subcore_barrier()

    def step(t, carry):
        avec = [sym_v[t, pl.ds(k * L, L)] for k in range(SYM // L)]

        @plsc.parallel_loop(0, SRC)
        def src_body(s):
            grp = (s // L) * L
            lane = s - grp
            sv = st_v[pl.ds(grp, L)]
            splat = jnp.take_along_axis(sv, jnp.broadcast_to(lane, (L,)),
                                        axis=0)
            off = s * (SYM // 2)
            for k in range(SYM // (2 * L)):
                pair = dfa_v[pl.ds(off + k * L, L)]
                lo = pair & jnp.int32(0xFFFF)
                hi = lax.shift_right_logical(pair, jnp.int32(16))
                plsc.addupdate_scatter(acc_v, [lo], splat * avec[2 * k])
                plsc.addupdate_scatter(acc_v, [hi], splat * avec[2 * k + 1])

        parity = lax.bitwise_and(t, 1)
        pltpu.sync_copy(acc_v, part_sh.at[parity, sid])

        @plsc.parallel_loop(0, S // L, unroll=8)
        def zero_acc(i):
            acc_v[pl.ds(i * L, L)] = jnp.zeros((L,), jnp.float32)

        plsc.subcore_barrier()
        pltpu.sync_copy(part_sh.at[parity, :, pl.ds(base, SRC)], red_v)

        @plsc.parallel_loop(0, SRC // L)
        def red_body(i):
            tot = red_v[0, pl.ds(i * L, L)]
            for r in range(1, NS):
                tot = tot + red_v[r, pl.ds(i * L, L)]
            st_v[pl.ds(i * L, L)] = tot
        return carry

    lax.fori_loop(0, T, step, None)

    pltpu.sync_copy(st_v, out_hbm.at[pl.ds(base, SRC)])


@jax.jit
def kernel(symbols_prob, dfa_table):
    d = dfa_table.astype(jnp.uint32).reshape(S, SYM // (2 * L), 2, L)
    dfa_packed = (d[:, :, 0, :] | (d[:, :, 1, :] << 16)).astype(
        jnp.int32).reshape(-1)
    mesh = plsc.VectorSubcoreMesh(core_axis_name="c", subcore_axis_name="s",
                                  num_cores=1, num_subcores=NS)
    run = pl.kernel(
        _fuzzy_body,
        out_type=jax.ShapeDtypeStruct((S,), jnp.float32),
        mesh=mesh,
        compiler_params=pltpu.CompilerParams(needs_layout_passes=False),
        scratch_types=[
            pltpu.VMEM((PW,), jnp.int32),
            pltpu.VMEM((T, SYM), jnp.float32),
            pltpu.VMEM((SRC,), jnp.float32),
            pltpu.VMEM((S,), jnp.float32),
            pltpu.VMEM((NS, SRC), jnp.float32),
            pltpu.VMEM_SHARED((2, NS, S), jnp.float32),
        ],
    )
    return run(symbols_prob, dfa_packed)
